# initial kernel scaffold (unmeasured)
import jax
import jax.numpy as jnp
from jax import lax
from jax.experimental import pallas as pl
from jax.experimental.pallas import tpu as pltpu


def kernel(
    x,
):
    def body(*refs):
        pass

    out_shape = jax.ShapeDtypeStruct(..., jnp.float32)
    return pl.pallas_call(body, out_shape=out_shape)(...)


# baseline (device time: 33349 ns/iter reference)
import jax
import jax.numpy as jnp
from jax import lax
from jax.experimental import pallas as pl
from jax.experimental.pallas import tpu as pltpu

N_DEV = 8
BLOCK = 512


def _colwise_prod(t):
    rows = t.shape[0]
    while rows > 1:
        half = rows // 2
        t = t[:half, :] * t[half : 2 * half, :]
        rows = half
    return t


def _block_cumprod(c):
    rows, n = c.shape
    s = 1
    while s < rows:
        shifted = jnp.concatenate(
            [jnp.ones((s, n), jnp.float32), c[: rows - s, :]], axis=0
        )
        c = c * shifted
        s *= 2
    return c


def kernel(x):
    m, n = x.shape
    assert m % BLOCK == 0

    def body(x_ref, out_ref, my_tot_ref, totals_ref, send_sems, recv_sems):
        my = lax.axis_index("i")

        my_tot_ref[:, :] = _colwise_prod(x_ref[:, :])

        for j in range(N_DEV):

            @pl.when(my != j)
            def _():
                send = pltpu.make_async_remote_copy(
                    src_ref=my_tot_ref,
                    dst_ref=totals_ref.at[my],
                    send_sem=send_sems.at[j],
                    recv_sem=recv_sems.at[my],
                    device_id=(j,),
                    device_id_type=pl.DeviceIdType.MESH,
                )
                send.start()

        for j in range(N_DEV):

            @pl.when(my != j)
            def _():
                send = pltpu.make_async_remote_copy(
                    src_ref=my_tot_ref,
                    dst_ref=totals_ref.at[my],
                    send_sem=send_sems.at[j],
                    recv_sem=recv_sems.at[my],
                    device_id=(j,),
                    device_id_type=pl.DeviceIdType.MESH,
                )
                send.wait_send()
                recv = pltpu.make_async_remote_copy(
                    src_ref=my_tot_ref,
                    dst_ref=totals_ref.at[j],
                    send_sem=send_sems.at[j],
                    recv_sem=recv_sems.at[j],
                    device_id=(j,),
                    device_id_type=pl.DeviceIdType.MESH,
                )
                recv.wait_recv()

        pref = jnp.ones((1, n), jnp.float32)
        for j in range(N_DEV - 1):
            pref = pref * jnp.where(j < my, totals_ref[j, :, :], 1.0)

        def blk(b, carry):
            xb = x_ref[pl.ds(b * BLOCK, BLOCK), :]
            cb = _block_cumprod(xb) * carry
            out_ref[pl.ds(b * BLOCK, BLOCK), :] = cb
            return cb[BLOCK - 1 : BLOCK, :]

        lax.fori_loop(0, m // BLOCK, blk, pref)

    return pl.pallas_call(
        body,
        out_shape=jax.ShapeDtypeStruct((m, n), jnp.float32),
        in_specs=[pl.BlockSpec(memory_space=pltpu.VMEM)],
        out_specs=pl.BlockSpec(memory_space=pltpu.VMEM),
        scratch_shapes=[
            pltpu.VMEM((1, n), jnp.float32),
            pltpu.VMEM((N_DEV, 1, n), jnp.float32),
            pltpu.SemaphoreType.DMA((N_DEV,)),
            pltpu.SemaphoreType.DMA((N_DEV,)),
        ],
    )(x)
